# baseline (speedup 1.0000x reference)
import functools
import math

import jax
import jax.numpy as jnp
from jax import lax
from jax.experimental import pallas as pl
from jax.experimental.pallas import tpu as pltpu
from jax.experimental.pallas import tpu_sc as plsc

EPS = 1e-5
NC, NS = 2, 16
NW = NC * NS
ROWS_BLK = 2048
CHUNK = 16
NBUF = 4
NBUF_O = 2


def _ln_table_body(pe_ref, w_ref, b_ref, out_ref, *, scale):
    r = pe_ref[...] * scale
    mu = jnp.mean(r, axis=-1, keepdims=True)
    d = r - mu
    var = jnp.mean(d * d, axis=-1, keepdims=True)
    ln = d * lax.rsqrt(var + EPS) * w_ref[...] + b_ref[...]
    bits = lax.bitcast_convert_type(ln, jnp.int32)
    rnd = lax.shift_right_logical(
        bits + 0x7FFF + lax.bitwise_and(lax.shift_right_logical(bits, 16), 1),
        16)
    half = ln.shape[1] // 2
    out_ref[...] = lax.bitwise_or(rnd[:, :half],
                                  lax.shift_left(rnd[:, half:], 16))


def _ln_table(pe, ln_w, ln_b):
    max_len, d_model = pe.shape
    scale = float(math.sqrt(d_model))
    grid = max_len // ROWS_BLK
    return pl.pallas_call(
        functools.partial(_ln_table_body, scale=scale),
        grid=(grid,),
        in_specs=[
            pl.BlockSpec((ROWS_BLK, d_model), lambda i: (i, 0)),
            pl.BlockSpec((1, d_model), lambda i: (0, 0)),
            pl.BlockSpec((1, d_model), lambda i: (0, 0)),
        ],
        out_specs=pl.BlockSpec((ROWS_BLK, d_model // 2), lambda i: (i, 0)),
        out_shape=jax.ShapeDtypeStruct((max_len, d_model // 2), jnp.int32),
    )(pe, ln_w.reshape(1, d_model), ln_b.reshape(1, d_model))


def _gather_add_body(n, d_model, pos_hbm, x_hbm, lnpe_hbm, out_hbm,
                     idx2, xb, rw, ob, sems):
    wid = lax.axis_index("s") * NC + lax.axis_index("c")
    per_w = n // NW
    nchunks = per_w // CHUNK
    ngroups = nchunks // NBUF
    nvec = d_model // 16
    sx, sg, so = sems

    pltpu.sync_copy(pos_hbm.at[pl.ds(wid * nchunks, nchunks)], idx2)

    def x_copy(c, p):
        base = wid * per_w + c * CHUNK
        return pltpu.make_async_copy(x_hbm.at[pl.ds(base, CHUNK)],
                                     xb.at[p], sx.at[p])

    def g_copy(c, p):
        return pltpu.make_async_copy(lnpe_hbm.at[idx2.at[c]],
                                     rw.at[p], sg.at[p])

    def o_copy(c, po):
        base = wid * per_w + c * CHUNK
        return pltpu.make_async_copy(ob.at[po], out_hbm.at[pl.ds(base, CHUNK)],
                                     so.at[po])

    for p in range(NBUF):
        x_copy(p, p).start()
        g_copy(p, p).start()

    def group(g, carry):
        for p in range(NBUF):
            c = g * NBUF + p
            x_copy(c, p).wait()
            g_copy(c, p).wait()

            po = p % NBUF_O
            @pl.when(c >= NBUF_O)
            def _():
                o_copy(c - NBUF_O, po).wait()

            def row_add(i, cc, p=p, po=po):
                half = d_model // 2
                for j in range(nvec // 2):
                    vi = rw[p, i, pl.ds(j * 16, 16)]
                    a = lax.bitcast_convert_type(
                        lax.shift_left(vi, 16), jnp.float32)
                    b = lax.bitcast_convert_type(
                        lax.bitwise_and(vi, jnp.int32(-65536)), jnp.float32)
                    sl_a = pl.ds(j * 16, 16)
                    sl_b = pl.ds(half + j * 16, 16)
                    ob[po, i, sl_a] = xb[p, i, sl_a] + a
                    ob[po, i, sl_b] = xb[p, i, sl_b] + b
                return cc

            lax.fori_loop(0, CHUNK, row_add, 0)
            o_copy(c, po).start()

            cp = c + NBUF

            @pl.when(cp < nchunks)
            def _():
                g_copy(cp, p).start()
                x_copy(cp, p).start()
        return carry

    lax.fori_loop(0, ngroups, group, 0)
    for k in range(NBUF_O):
        c = nchunks - NBUF_O + k
        o_copy(c, (c % NBUF) % NBUF_O).wait()


def _gather_add(pos2, x2d, lnpe):
    n, d_model = x2d.shape
    mesh = plsc.VectorSubcoreMesh(core_axis_name="c", subcore_axis_name="s")
    nchunks = n // NW // CHUNK
    kern = functools.partial(_gather_add_body, n, d_model)
    return pl.kernel(
        kern,
        out_type=jax.ShapeDtypeStruct((n, d_model), jnp.float32),
        mesh=mesh,
        scratch_types=[
            pltpu.VMEM((nchunks, CHUNK), jnp.int32),
            pltpu.VMEM((NBUF, CHUNK, d_model), jnp.float32),
            pltpu.VMEM((NBUF, CHUNK, d_model // 2), jnp.int32),
            pltpu.VMEM((NBUF_O, CHUNK, d_model), jnp.float32),
            [pltpu.SemaphoreType.DMA((NBUF,)),
             pltpu.SemaphoreType.DMA((NBUF,)),
             pltpu.SemaphoreType.DMA((NBUF_O,))],
        ],
    )(pos2, x2d, lnpe)


def kernel(x, positions, pe, ln_w, ln_b):
    b, t, d_model = x.shape
    lnpe = _ln_table(pe, ln_w, ln_b)
    pos2 = positions.reshape(b * t // CHUNK, CHUNK).astype(jnp.int32)
    out2d = _gather_add(pos2, x.reshape(b * t, d_model), lnpe)
    return out2d.reshape(b, t, d_model)


# speedup vs baseline: 1.7759x; 1.7759x over previous
import functools
import math

import jax
import jax.numpy as jnp
from jax import lax
from jax.experimental import pallas as pl
from jax.experimental.pallas import tpu as pltpu
from jax.experimental.pallas import tpu_sc as plsc

EPS = 1e-5
NC, NS = 2, 16
NW = NC * NS
ROWS_BLK = 2048
CHUNK = 8
NBUF = 8
NBUF_O = 4


def _ln_table_body(pe_ref, w_ref, b_ref, out_ref, *, scale):
    r = pe_ref[...] * scale
    mu = jnp.mean(r, axis=-1, keepdims=True)
    d = r - mu
    var = jnp.mean(d * d, axis=-1, keepdims=True)
    ln = d * lax.rsqrt(var + EPS) * w_ref[...] + b_ref[...]
    bits = lax.bitcast_convert_type(ln, jnp.int32)
    rnd = lax.shift_right_logical(
        bits + 0x7FFF + lax.bitwise_and(lax.shift_right_logical(bits, 16), 1),
        16)
    half = ln.shape[1] // 2
    out_ref[...] = lax.bitwise_or(rnd[:, :half],
                                  lax.shift_left(rnd[:, half:], 16))


def _ln_table(pe, ln_w, ln_b):
    max_len, d_model = pe.shape
    scale = float(math.sqrt(d_model))
    grid = max_len // ROWS_BLK
    return pl.pallas_call(
        functools.partial(_ln_table_body, scale=scale),
        grid=(grid,),
        in_specs=[
            pl.BlockSpec((ROWS_BLK, d_model), lambda i: (i, 0)),
            pl.BlockSpec((1, d_model), lambda i: (0, 0)),
            pl.BlockSpec((1, d_model), lambda i: (0, 0)),
        ],
        out_specs=pl.BlockSpec((ROWS_BLK, d_model // 2), lambda i: (i, 0)),
        out_shape=jax.ShapeDtypeStruct((max_len, d_model // 2), jnp.int32),
    )(pe, ln_w.reshape(1, d_model), ln_b.reshape(1, d_model))


def _gather_add_body(n, d_model, pos_hbm, x_hbm, lnpe_hbm, out_hbm,
                     idx2, xb, rw, ob, sems):
    wid = lax.axis_index("s") * NC + lax.axis_index("c")
    per_w = n // NW
    nchunks = per_w // CHUNK
    ngroups = nchunks // NBUF
    nvec = d_model // 16
    sx, sg, so = sems

    pltpu.sync_copy(pos_hbm.at[pl.ds(wid * nchunks, nchunks)], idx2)

    def x_copy(c, p):
        base = wid * per_w + c * CHUNK
        return pltpu.make_async_copy(x_hbm.at[pl.ds(base, CHUNK)],
                                     xb.at[p], sx.at[p])

    def g_copy(c, p):
        return pltpu.make_async_copy(lnpe_hbm.at[idx2.at[c]],
                                     rw.at[p], sg.at[p])

    def o_copy(c, po):
        base = wid * per_w + c * CHUNK
        return pltpu.make_async_copy(ob.at[po], out_hbm.at[pl.ds(base, CHUNK)],
                                     so.at[po])

    for p in range(NBUF):
        x_copy(p, p).start()
        g_copy(p, p).start()

    def group(g, carry):
        for p in range(NBUF):
            c = g * NBUF + p
            x_copy(c, p).wait()
            g_copy(c, p).wait()

            po = p % NBUF_O
            @pl.when(c >= NBUF_O)
            def _():
                o_copy(c - NBUF_O, po).wait()

            def row_add(i, cc, p=p, po=po):
                half = d_model // 2
                for j in range(nvec // 2):
                    vi = rw[p, i, pl.ds(j * 16, 16)]
                    a = lax.bitcast_convert_type(
                        lax.shift_left(vi, 16), jnp.float32)
                    b = lax.bitcast_convert_type(
                        lax.bitwise_and(vi, jnp.int32(-65536)), jnp.float32)
                    sl_a = pl.ds(j * 16, 16)
                    sl_b = pl.ds(half + j * 16, 16)
                    ob[po, i, sl_a] = xb[p, i, sl_a] + a
                    ob[po, i, sl_b] = xb[p, i, sl_b] + b
                return cc

            lax.fori_loop(0, CHUNK, row_add, 0)
            o_copy(c, po).start()

            cp = c + NBUF

            @pl.when(cp < nchunks)
            def _():
                g_copy(cp, p).start()
                x_copy(cp, p).start()
        return carry

    lax.fori_loop(0, ngroups, group, 0)
    for k in range(NBUF_O):
        c = nchunks - NBUF_O + k
        o_copy(c, (c % NBUF) % NBUF_O).wait()


def _gather_add(pos2, x2d, lnpe):
    n, d_model = x2d.shape
    mesh = plsc.VectorSubcoreMesh(core_axis_name="c", subcore_axis_name="s")
    nchunks = n // NW // CHUNK
    kern = functools.partial(_gather_add_body, n, d_model)
    return pl.kernel(
        kern,
        out_type=jax.ShapeDtypeStruct((n, d_model), jnp.float32),
        mesh=mesh,
        scratch_types=[
            pltpu.VMEM((nchunks, CHUNK), jnp.int32),
            pltpu.VMEM((NBUF, CHUNK, d_model), jnp.float32),
            pltpu.VMEM((NBUF, CHUNK, d_model // 2), jnp.int32),
            pltpu.VMEM((NBUF_O, CHUNK, d_model), jnp.float32),
            [pltpu.SemaphoreType.DMA((NBUF,)),
             pltpu.SemaphoreType.DMA((NBUF,)),
             pltpu.SemaphoreType.DMA((NBUF_O,))],
        ],
    )(pos2, x2d, lnpe)


def kernel(x, positions, pe, ln_w, ln_b):
    b, t, d_model = x.shape
    lnpe = _ln_table(pe, ln_w, ln_b)
    pos2 = positions.reshape(b * t // CHUNK, CHUNK).astype(jnp.int32)
    out2d = _gather_add(pos2, x.reshape(b * t, d_model), lnpe)
    return out2d.reshape(b, t, d_model)
